# Initial kernel scaffold; baseline (speedup 1.0000x reference)
import jax
import jax.numpy as jnp
from jax.experimental import pallas as pl


def kernel(x, table):
    raise NotImplementedError("write your pallas kernel here")


# speedup vs baseline: 1.7477x; 1.7477x over previous
import math

import jax
import jax.numpy as jnp
from jax import lax
from jax.experimental import pallas as pl
from jax.experimental.pallas import tpu as pltpu
from jax.experimental.pallas import tpu_sc as plsc

D_MODEL = 128
SCALE = math.sqrt(float(D_MODEL))

_INFO = plsc.get_sparse_core_info()
NC = _INFO.num_cores
NS = _INFO.num_subcores
NW = NC * NS
LANES = _INFO.num_lanes

CHUNK = 128
NBUF = 5


def _emb_body(x_hbm, table_hbm, out_hbm,
              idx_v, b0, b1, b2, b3, b4,
              gs0, gs1, gs2, gs3, gs4, os0, os1, os2, os3, os4):
    bufs = (b0, b1, b2, b3, b4)
    gsem = (gs0, gs1, gs2, gs3, gs4)
    osem = (os0, os1, os2, os3, os4)

    n_chunks = x_hbm.shape[1]
    wid = lax.axis_index("s") * NC + lax.axis_index("c")
    base = wid * (n_chunks * CHUNK)

    pltpu.sync_copy(x_hbm.at[wid], idx_v)

    for b in range(NBUF):
        pltpu.async_copy(table_hbm.at[idx_v.at[b]], bufs[b], gsem[b])

    def scale_chunk(buf):
        def row(r, _):
            for c in range(D_MODEL // LANES):
                sl = pl.ds(c * LANES, LANES)
                buf[r, sl] = buf[r, sl] * SCALE
            return 0
        lax.fori_loop(0, CHUNK, row, 0, unroll=2)

    def group(g, _):
        for b in range(NBUF):
            j = g * NBUF + b
            pltpu.make_async_copy(table_hbm.at[idx_v.at[j]], bufs[b],
                                  gsem[b]).wait()
            scale_chunk(bufs[b])
            out_slice = out_hbm.at[pl.ds(base + j * CHUNK, CHUNK)]
            pltpu.async_copy(bufs[b], out_slice, osem[b])

            @pl.when(j + NBUF < n_chunks)
            def _():
                pltpu.make_async_copy(bufs[b], out_slice, osem[b]).wait()
                pltpu.async_copy(table_hbm.at[idx_v.at[j + NBUF]],
                                 bufs[b], gsem[b])
        return 0

    n_groups = n_chunks // NBUF
    lax.fori_loop(0, n_groups, group, 0)

    for b in range(NBUF):
        j = (n_groups - 1) * NBUF + b
        pltpu.make_async_copy(bufs[b],
                              out_hbm.at[pl.ds(base + j * CHUNK, CHUNK)],
                              osem[b]).wait()


def kernel(x, table):
    bsz, seq = x.shape
    total = bsz * seq
    assert total % (NW * CHUNK) == 0
    n_chunks = total // (NW * CHUNK)
    assert n_chunks % NBUF == 0

    xw = x.reshape(NW, n_chunks, CHUNK).astype(jnp.int32)

    mesh = plsc.VectorSubcoreMesh(core_axis_name="c", subcore_axis_name="s")
    run = pl.kernel(
        _emb_body,
        out_type=jax.ShapeDtypeStruct((total, D_MODEL), jnp.float32),
        mesh=mesh,
        scratch_types=(
            [pltpu.VMEM((n_chunks, CHUNK), jnp.int32)]
            + [pltpu.VMEM((CHUNK, D_MODEL), jnp.float32)] * NBUF
            + [pltpu.SemaphoreType.DMA] * (2 * NBUF)
        ),
    )
    out = run(xw, table)
    return out.reshape(bsz, seq, D_MODEL)


# speedup vs baseline: 1.7676x; 1.0113x over previous
import math

import jax
import jax.numpy as jnp
from jax import lax
from jax.experimental import pallas as pl
from jax.experimental.pallas import tpu as pltpu
from jax.experimental.pallas import tpu_sc as plsc

D_MODEL = 128
SCALE = math.sqrt(float(D_MODEL))

_INFO = plsc.get_sparse_core_info()
NC = _INFO.num_cores
NS = _INFO.num_subcores
NW = NC * NS
LANES = _INFO.num_lanes

CHUNK = 128
NBUF = 5
DEPTH = 3


def _emb_body(x_hbm, table_hbm, out_hbm,
              idx_v, b0, b1, b2, b3, b4,
              gs0, gs1, gs2, gs3, gs4, os0, os1, os2, os3, os4):
    bufs = (b0, b1, b2, b3, b4)
    gsem = (gs0, gs1, gs2, gs3, gs4)
    osem = (os0, os1, os2, os3, os4)

    n_chunks = x_hbm.shape[1]
    wid = lax.axis_index("s") * NC + lax.axis_index("c")
    base = wid * (n_chunks * CHUNK)

    pltpu.sync_copy(x_hbm.at[wid], idx_v)

    for b in range(DEPTH):
        pltpu.async_copy(table_hbm.at[idx_v.at[b]], bufs[b], gsem[b])

    def scale_chunk(buf):
        def row(r, _):
            for c in range(D_MODEL // LANES):
                sl = pl.ds(c * LANES, LANES)
                buf[r, sl] = buf[r, sl] * SCALE
            return 0
        lax.fori_loop(0, CHUNK, row, 0, unroll=2)

    def out_slice(j):
        return out_hbm.at[pl.ds(base + j * CHUNK, CHUNK)]

    def group(g, _):
        for b in range(NBUF):
            j = g * NBUF + b
            bp = (b + DEPTH) % NBUF

            @pl.when(j + DEPTH < n_chunks)
            def _():
                @pl.when(j >= NBUF - DEPTH)
                def _():
                    jprev = j - (NBUF - DEPTH)
                    pltpu.make_async_copy(bufs[bp], out_slice(jprev),
                                          osem[bp]).wait()
                pltpu.async_copy(table_hbm.at[idx_v.at[j + DEPTH]],
                                 bufs[bp], gsem[bp])

            pltpu.make_async_copy(table_hbm.at[idx_v.at[j]], bufs[b],
                                  gsem[b]).wait()
            scale_chunk(bufs[b])
            pltpu.async_copy(bufs[b], out_slice(j), osem[b])
        return 0

    n_groups = n_chunks // NBUF
    lax.fori_loop(0, n_groups, group, 0)

    for i in range(NBUF):
        j = n_chunks - NBUF + i
        pltpu.make_async_copy(bufs[j % NBUF], out_slice(j),
                              osem[j % NBUF]).wait()


def kernel(x, table):
    bsz, seq = x.shape
    total = bsz * seq
    assert total % (NW * CHUNK) == 0
    n_chunks = total // (NW * CHUNK)
    assert n_chunks % NBUF == 0

    xw = x.reshape(NW, n_chunks, CHUNK).astype(jnp.int32)

    mesh = plsc.VectorSubcoreMesh(core_axis_name="c", subcore_axis_name="s")
    run = pl.kernel(
        _emb_body,
        out_type=jax.ShapeDtypeStruct((total, D_MODEL), jnp.float32),
        mesh=mesh,
        scratch_types=(
            [pltpu.VMEM((n_chunks, CHUNK), jnp.int32)]
            + [pltpu.VMEM((CHUNK, D_MODEL), jnp.float32)] * NBUF
            + [pltpu.SemaphoreType.DMA] * (2 * NBUF)
        ),
    )
    out = run(xw, table)
    return out.reshape(bsz, seq, D_MODEL)


# speedup vs baseline: 1.7704x; 1.0016x over previous
import math

import jax
import jax.numpy as jnp
from jax import lax
from jax.experimental import pallas as pl
from jax.experimental.pallas import tpu as pltpu
from jax.experimental.pallas import tpu_sc as plsc

D_MODEL = 128
SCALE = math.sqrt(float(D_MODEL))

_INFO = plsc.get_sparse_core_info()
NC = _INFO.num_cores
NS = _INFO.num_subcores
NW = NC * NS
LANES = _INFO.num_lanes

CHUNK = 128
NBUF = 5
DEPTH = 3


def _emb_body(x_hbm, table_hbm, out_hbm,
              idx_v, b0, b1, b2, b3, b4,
              gs0, gs1, gs2, gs3, gs4, os0, os1, os2, os3, os4):
    bufs = (b0, b1, b2, b3, b4)
    gsem = (gs0, gs1, gs2, gs3, gs4)
    osem = (os0, os1, os2, os3, os4)

    n_chunks = x_hbm.shape[1]
    wid = lax.axis_index("s") * NC + lax.axis_index("c")
    base = wid * (n_chunks * CHUNK)

    pltpu.sync_copy(x_hbm.at[wid], idx_v)

    for b in range(DEPTH):
        pltpu.async_copy(table_hbm.at[idx_v.at[b]], bufs[b], gsem[b])

    def scale_chunk(buf):
        @plsc.parallel_loop(0, CHUNK, step=1, unroll=4)
        def _(r):
            for c in range(D_MODEL // LANES):
                sl = pl.ds(c * LANES, LANES)
                buf[r, sl] = buf[r, sl] * SCALE

    def out_slice(j):
        return out_hbm.at[pl.ds(base + j * CHUNK, CHUNK)]

    def group(g, _):
        for b in range(NBUF):
            j = g * NBUF + b
            bp = (b + DEPTH) % NBUF

            @pl.when(j + DEPTH < n_chunks)
            def _():
                @pl.when(j >= NBUF - DEPTH)
                def _():
                    jprev = j - (NBUF - DEPTH)
                    pltpu.make_async_copy(bufs[bp], out_slice(jprev),
                                          osem[bp]).wait()
                pltpu.async_copy(table_hbm.at[idx_v.at[j + DEPTH]],
                                 bufs[bp], gsem[bp])

            pltpu.make_async_copy(table_hbm.at[idx_v.at[j]], bufs[b],
                                  gsem[b]).wait()
            scale_chunk(bufs[b])
            pltpu.async_copy(bufs[b], out_slice(j), osem[b])
        return 0

    n_groups = n_chunks // NBUF
    lax.fori_loop(0, n_groups, group, 0)

    for i in range(NBUF):
        j = n_chunks - NBUF + i
        pltpu.make_async_copy(bufs[j % NBUF], out_slice(j),
                              osem[j % NBUF]).wait()


def kernel(x, table):
    bsz, seq = x.shape
    total = bsz * seq
    assert total % (NW * CHUNK) == 0
    n_chunks = total // (NW * CHUNK)
    assert n_chunks % NBUF == 0

    xw = x.reshape(NW, n_chunks, CHUNK).astype(jnp.int32)

    mesh = plsc.VectorSubcoreMesh(core_axis_name="c", subcore_axis_name="s")
    run = pl.kernel(
        _emb_body,
        out_type=jax.ShapeDtypeStruct((total, D_MODEL), jnp.float32),
        mesh=mesh,
        scratch_types=(
            [pltpu.VMEM((n_chunks, CHUNK), jnp.int32)]
            + [pltpu.VMEM((CHUNK, D_MODEL), jnp.float32)] * NBUF
            + [pltpu.SemaphoreType.DMA] * (2 * NBUF)
        ),
    )
    out = run(xw, table)
    return out.reshape(bsz, seq, D_MODEL)
